# baseline (speedup 1.0000x reference)
import functools

import jax
import jax.numpy as jnp
from jax import lax
from jax.experimental import pallas as pl
from jax.experimental.pallas import tpu as pltpu
from jax.experimental.pallas import tpu_sc as plsc

_NUM_CORES = 2
_NUM_SUBCORES = 16
_NW = _NUM_CORES * _NUM_SUBCORES
_LANES = 16


@functools.lru_cache(maxsize=None)
def _make_gather(BSZ: int, HIST: int, V: int, D: int):
    assert BSZ % 128 == 0 and D % 8 == 0
    n_bt = BSZ // 128
    n_eg = D // 8
    n_units = HIST * n_bt
    assert n_units % (2 * _NW) == 0
    u_per_w = n_units // _NW

    mesh = plsc.VectorSubcoreMesh(
        core_axis_name="c", subcore_axis_name="s",
        num_cores=_NUM_CORES, num_subcores=_NUM_SUBCORES)

    @functools.partial(
        pl.kernel,
        mesh=mesh,
        compiler_params=pltpu.CompilerParams(
            use_tc_tiling_on_sc=False, needs_layout_passes=False),
        out_type=jax.ShapeDtypeStruct((HIST, n_eg, n_bt, 8, 128), jnp.float32),
        scratch_types=[
            [pltpu.VMEM((128,), jnp.int32) for _ in range(2)],
            [pltpu.VMEM((128, D), jnp.float32) for _ in range(2)],
            [pltpu.VMEM((n_eg, 8, 128), jnp.float32) for _ in range(2)],
            [pltpu.SemaphoreType.DMA for _ in range(2)],
            [pltpu.SemaphoreType.DMA for _ in range(2)],
            [pltpu.SemaphoreType.DMA for _ in range(2)],
        ],
    )
    def gather_kernel(idx_hbm, table_hbm, out_hbm, idxb, G, T, isems, gsems, wsems):
        wid = lax.axis_index("s") * _NUM_CORES + lax.axis_index("c")
        u0 = wid * u_per_w

        def hbt(u):
            ug = u0 + u
            return ug // n_bt, ug % n_bt

        def i_desc(u, b):
            h, bt = hbt(u)
            return pltpu.make_async_copy(
                idx_hbm.at[pl.ds(h * BSZ + bt * 128, 128)], idxb[b], isems[b])

        def g_desc(u, b):
            return pltpu.make_async_copy(
                table_hbm.at[idxb[b]], G[b], gsems[b])

        def w_desc(u, b):
            h, bt = hbt(u)
            return pltpu.make_async_copy(T[b], out_hbm.at[h, :, bt], wsems[b])

        row_vecs = [lax.iota(jnp.int32, _LANES) + j * _LANES
                    for j in range(128 // _LANES)]

        def transpose(b):
            for e in range(D):
                col = jnp.full((_LANES,), e, dtype=jnp.int32)
                for j, row in enumerate(row_vecs):
                    v = plsc.load_gather(G[b], [row, col])
                    T[b][e // 8, e % 8, pl.ds(j * _LANES, _LANES)] = v

        i_desc(0, 0).start()
        i_desc(1, 1).start()
        i_desc(0, 0).wait()
        g_desc(0, 0).start()

        def outer(o, carry):
            for b in range(2):
                u = o * 2 + b
                g_desc(u, b).wait()

                @pl.when(u + 2 < u_per_w)
                def _():
                    i_desc(u + 2, b).start()

                @pl.when(u + 1 < u_per_w)
                def _():
                    i_desc(u + 1, 1 - b).wait()
                    g_desc(u + 1, 1 - b).start()

                @pl.when(u >= 2)
                def _():
                    w_desc(u - 2, b).wait()

                transpose(b)
                w_desc(u, b).start()
            return carry

        lax.fori_loop(0, u_per_w // 2, outer, 0)
        w_desc(u_per_w - 2, 0).wait()
        w_desc(u_per_w - 1, 1).wait()

    return gather_kernel


@jax.jit
def kernel(x, weight):
    bsz, hist = x.shape
    V, D = weight.shape
    flat = x.T.reshape(bsz * hist).astype(jnp.int32)
    o5 = _make_gather(bsz, hist, V, D)(flat, weight)
    return o5.transpose(2, 4, 0, 1, 3).reshape(bsz, hist, D)


# speedup vs baseline: 1.0159x; 1.0159x over previous
import functools

import jax
import jax.numpy as jnp
from jax import lax
from jax.experimental import pallas as pl
from jax.experimental.pallas import tpu as pltpu
from jax.experimental.pallas import tpu_sc as plsc

_NUM_CORES = 2
_NUM_SUBCORES = 16
_NW = _NUM_CORES * _NUM_SUBCORES

_CHUNK = 640
_NBUF = 4
_LOOKAHEAD = 2

_VBLK = 8192


@functools.lru_cache(maxsize=None)
def _make_table_transpose(V: int, D: int):
    grid = (V + _VBLK - 1) // _VBLK

    def body(wt_ref, out_ref):
        out_ref[...] = wt_ref[...].T

    return pl.pallas_call(
        body,
        grid=(grid,),
        in_specs=[pl.BlockSpec((D, _VBLK), lambda i: (0, i))],
        out_specs=pl.BlockSpec((_VBLK, D), lambda i: (i, 0)),
        out_shape=jax.ShapeDtypeStruct((V, D), jnp.float32),
    )


@functools.lru_cache(maxsize=None)
def _make_out_transpose(BSZ: int, HIST: int, D: int):
    n_bt = BSZ // 128
    n_eg = D // 8

    def body(rows_ref, out_ref):
        t = rows_ref[0].T.reshape(n_eg, 8, n_bt, 128)
        out_ref[...] = t.transpose(0, 2, 1, 3)[None]

    return pl.pallas_call(
        body,
        grid=(HIST,),
        in_specs=[pl.BlockSpec((1, BSZ, D), lambda h: (h, 0, 0))],
        out_specs=pl.BlockSpec((1, n_eg, n_bt, 8, 128),
                               lambda h: (h, 0, 0, 0, 0)),
        out_shape=jax.ShapeDtypeStruct((HIST, n_eg, n_bt, 8, 128),
                                       jnp.float32),
    )


@functools.lru_cache(maxsize=None)
def _make_gather(B: int, V: int, D: int):
    assert B % _NW == 0
    b_per_w = B // _NW
    chunk, nbuf, k = _CHUNK, _NBUF, _LOOKAHEAD
    assert b_per_w % chunk == 0
    n_chunks = b_per_w // chunk
    assert n_chunks % nbuf == 0 and nbuf > k

    mesh = plsc.VectorSubcoreMesh(
        core_axis_name="c", subcore_axis_name="s",
        num_cores=_NUM_CORES, num_subcores=_NUM_SUBCORES)

    @functools.partial(
        pl.kernel,
        mesh=mesh,
        compiler_params=pltpu.CompilerParams(use_tc_tiling_on_sc=False),
        out_type=jax.ShapeDtypeStruct((B, D), jnp.float32),
        scratch_types=[
            pltpu.VMEM((b_per_w,), jnp.int32),
            [pltpu.VMEM((chunk, D), jnp.float32) for _ in range(nbuf)],
            [pltpu.SemaphoreType.DMA for _ in range(nbuf)],
            [pltpu.SemaphoreType.DMA for _ in range(nbuf)],
        ],
    )
    def gather_kernel(idx_hbm, table_hbm, out_hbm, idx_v, rows, gsems, wsems):
        wid = lax.axis_index("s") * _NUM_CORES + lax.axis_index("c")
        base = wid * b_per_w
        pltpu.sync_copy(idx_hbm.at[pl.ds(base, b_per_w)], idx_v)

        def g_desc(c, b):
            return pltpu.make_async_copy(
                table_hbm.at[idx_v.at[pl.ds(c * chunk, chunk)]],
                rows[b], gsems[b])

        def w_desc(c, b):
            return pltpu.make_async_copy(
                rows[b], out_hbm.at[pl.ds(base + c * chunk, chunk)], wsems[b])

        for b in range(k):
            g_desc(b, b).start()

        def outer(o, carry):
            for b in range(nbuf):
                c = o * nbuf + b
                pb = (b + k) % nbuf
                pc = c + k

                @pl.when(pc < n_chunks)
                def _(pc=pc, pb=pb):
                    @pl.when(pc >= nbuf)
                    def _():
                        w_desc(pc - nbuf, pb).wait()
                    g_desc(pc, pb).start()

                g_desc(c, b).wait()
                w_desc(c, b).start()
            return carry

        lax.fori_loop(0, n_chunks // nbuf, outer, 0)
        for b in range(nbuf):
            w_desc(n_chunks - nbuf + b, b).wait()

    return gather_kernel


@jax.jit
def kernel(x, weight):
    bsz, hist = x.shape
    V, D = weight.shape
    flat = x.T.reshape(bsz * hist).astype(jnp.int32)
    table = _make_table_transpose(V, D)(weight.T)
    rows = _make_gather(bsz * hist, V, D)(flat, table)
    o5 = _make_out_transpose(bsz, hist, D)(rows.reshape(hist, bsz, D))
    return o5.transpose(2, 4, 0, 1, 3).reshape(bsz, hist, D)


# speedup vs baseline: 1.1363x; 1.1185x over previous
import functools

import jax
import jax.numpy as jnp
from jax import lax
from jax.experimental import pallas as pl
from jax.experimental.pallas import tpu as pltpu
from jax.experimental.pallas import tpu_sc as plsc

_NUM_CORES = 2
_NUM_SUBCORES = 16
_NW = _NUM_CORES * _NUM_SUBCORES

_CHUNK = 640
_NBUF = 4
_LOOKAHEAD = 2

_VBLK = 32768
_HBLK = 4


@functools.lru_cache(maxsize=None)
def _make_table_transpose(V: int, D: int):
    grid = (V + _VBLK - 1) // _VBLK

    def body(wt_ref, out_ref):
        out_ref[...] = wt_ref[...].T

    return pl.pallas_call(
        body,
        grid=(grid,),
        in_specs=[pl.BlockSpec((D, _VBLK), lambda i: (0, i))],
        out_specs=pl.BlockSpec((_VBLK, D), lambda i: (i, 0)),
        out_shape=jax.ShapeDtypeStruct((V, D), jnp.float32),
    )


@functools.lru_cache(maxsize=None)
def _make_out_transpose(BSZ: int, HIST: int, D: int):
    n_bt = BSZ // 128
    n_eg = D // 8

    def body(rows_ref, out_ref):
        for i in range(_HBLK):
            t = rows_ref[i].T.reshape(n_eg, 8, n_bt, 128)
            out_ref[i] = t.transpose(0, 2, 1, 3)

    return pl.pallas_call(
        body,
        grid=(HIST // _HBLK,),
        in_specs=[pl.BlockSpec((_HBLK, BSZ, D), lambda h: (h, 0, 0))],
        out_specs=pl.BlockSpec((_HBLK, n_eg, n_bt, 8, 128),
                               lambda h: (h, 0, 0, 0, 0)),
        out_shape=jax.ShapeDtypeStruct((HIST, n_eg, n_bt, 8, 128),
                                       jnp.float32),
    )


@functools.lru_cache(maxsize=None)
def _make_gather(B: int, V: int, D: int):
    assert B % _NW == 0
    b_per_w = B // _NW
    chunk, nbuf, k = _CHUNK, _NBUF, _LOOKAHEAD
    assert b_per_w % chunk == 0
    n_chunks = b_per_w // chunk
    assert n_chunks % nbuf == 0 and nbuf > k

    mesh = plsc.VectorSubcoreMesh(
        core_axis_name="c", subcore_axis_name="s",
        num_cores=_NUM_CORES, num_subcores=_NUM_SUBCORES)

    @functools.partial(
        pl.kernel,
        mesh=mesh,
        compiler_params=pltpu.CompilerParams(use_tc_tiling_on_sc=False),
        out_type=jax.ShapeDtypeStruct((B, D), jnp.float32),
        scratch_types=[
            pltpu.VMEM((b_per_w,), jnp.int32),
            [pltpu.VMEM((chunk, D), jnp.float32) for _ in range(nbuf)],
            [pltpu.SemaphoreType.DMA for _ in range(nbuf)],
            [pltpu.SemaphoreType.DMA for _ in range(nbuf)],
        ],
    )
    def gather_kernel(idx_hbm, table_hbm, out_hbm, idx_v, rows, gsems, wsems):
        wid = lax.axis_index("s") * _NUM_CORES + lax.axis_index("c")
        base = wid * b_per_w
        pltpu.sync_copy(idx_hbm.at[pl.ds(base, b_per_w)], idx_v)

        def g_desc(c, b):
            return pltpu.make_async_copy(
                table_hbm.at[idx_v.at[pl.ds(c * chunk, chunk)]],
                rows[b], gsems[b])

        def w_desc(c, b):
            return pltpu.make_async_copy(
                rows[b], out_hbm.at[pl.ds(base + c * chunk, chunk)], wsems[b])

        for b in range(k):
            g_desc(b, b).start()

        def outer(o, carry):
            for b in range(nbuf):
                c = o * nbuf + b
                pb = (b + k) % nbuf
                pc = c + k

                @pl.when(pc < n_chunks)
                def _(pc=pc, pb=pb):
                    @pl.when(pc >= nbuf)
                    def _():
                        w_desc(pc - nbuf, pb).wait()
                    g_desc(pc, pb).start()

                g_desc(c, b).wait()
                w_desc(c, b).start()
            return carry

        lax.fori_loop(0, n_chunks // nbuf, outer, 0)
        for b in range(nbuf):
            w_desc(n_chunks - nbuf + b, b).wait()

    return gather_kernel


@jax.jit
def kernel(x, weight):
    bsz, hist = x.shape
    V, D = weight.shape
    flat = x.T.reshape(bsz * hist).astype(jnp.int32)
    table = _make_table_transpose(V, D)(weight.T)
    rows = _make_gather(bsz * hist, V, D)(flat, table)
    o5 = _make_out_transpose(bsz, hist, D)(rows.reshape(hist, bsz, D))
    return o5.transpose(2, 4, 0, 1, 3).reshape(bsz, hist, D)
